# baseline (speedup 1.0000x reference)
import functools

import jax
import jax.numpy as jnp
from jax import lax
from jax.experimental import pallas as pl
from jax.experimental.pallas import tpu as pltpu
from jax.experimental.pallas import tpu_sc as plsc

_N = 10000
_E = 320000
_D = 128
_H = 128
_NID = 1000
_NIDP = 1024
_MLP_H = 256
_C = 6

_BLK = 1000
_CH = 64
_NCORES = 2
_NSUB = 16
_NPAD = 10240
_RPT = _NPAD // _NSUB


def _stage_a1(x_ref, w_ref, m_ref):
    m_ref[...] = jnp.dot(x_ref[...], w_ref[...],
                         preferred_element_type=jnp.float32)


def _stage_a2(x_ref, w_ref, ids_ref, self0_ref, mask_ref):
    i = pl.program_id(0)
    x = x_ref[...]
    y = jnp.dot(x, w_ref[...], preferred_element_type=jnp.float32)
    s_t = y[:, 0:128]
    i_t = y[:, 128:256]
    rid = lax.broadcasted_iota(jnp.int32, (_BLK, 8, 128), 0) + i * _BLK
    hit = (rid == ids_ref[...][None, :, :]).astype(jnp.float32)
    maskf = jnp.minimum(jnp.sum(hit, axis=(1, 2)), 1.0)[:, None]
    self0_ref[...] = s_t + maskf * (i_t - s_t)
    mask_ref[...] = maskf


def _stage_b(self0_ref, p_ref, degt_ref, mask_ref, b0_ref, w_ref,
             self1_ref, m1_ref, dinv_ref):
    p = p_ref[...]
    agg = p[0] + p[1]
    deg = jnp.sum(degt_ref[...], axis=1, keepdims=True)
    dinv = 1.0 / jnp.maximum(deg, 1.0)
    h1 = jnp.maximum(self0_ref[...] + agg * dinv + b0_ref[...], 0.0)
    y = jnp.dot(h1, w_ref[...], preferred_element_type=jnp.float32)
    s_t = y[:, 0:128]
    i_t = y[:, 128:256]
    maskf = mask_ref[...]
    self1_ref[...] = s_t + maskf * (i_t - s_t)
    m1_ref[...] = y[:, 256:384]
    dinv_ref[...] = dinv


def _stage_d(self1_ref, p_ref, dinv_ref, b1_ref, wm1_ref, bm1_ref, wm2_ref, bm2_ref, out_ref):
    p = p_ref[...]
    agg = (p[0] + p[1]) * dinv_ref[...]
    h2 = jnp.maximum(self1_ref[...] + agg + b1_ref[...], 0.0)
    t = jnp.dot(h2, wm1_ref[...], preferred_element_type=jnp.float32)
    t = jnp.maximum(t + bm1_ref[...], 0.0)
    out_ref[...] = jnp.dot(t, wm2_ref[...], preferred_element_type=jnp.float32) + bm2_ref[...]


_NCHUNK = _E // _CH
_NW = _NCORES * _NSUB
_BUF = 4
def _make_segsum(with_deg, blkch):
    mesh = plsc.VectorSubcoreMesh(core_axis_name="c", subcore_axis_name="s")
    nblk_all = _NCHUNK // blkch
    extra = _NCHUNK - nblk_all * blkch

    part_t = jax.ShapeDtypeStruct((_NCORES, _NPAD, _H), jnp.float32)
    out_type = [part_t, jax.ShapeDtypeStruct((_NW, _NPAD), jnp.float32)] if with_deg else part_t
    scratch = [
        pltpu.VMEM((blkch, _CH), jnp.int32),
        pltpu.VMEM((blkch, _CH), jnp.int32),
        pltpu.VMEM((_BUF, _CH, _H), jnp.float32),
        pltpu.VMEM_SHARED((_NPAD, _H), jnp.float32),
    ] + [pltpu.SemaphoreType.DMA] * (_BUF + 1)
    if with_deg:
        scratch.append(pltpu.VMEM((_NPAD,), jnp.float32))

    @functools.partial(
        pl.kernel, out_type=out_type, mesh=mesh, scratch_types=scratch,
        compiler_params=pltpu.CompilerParams(needs_layout_passes=False))
    def seg(*refs):
        if with_deg:
            table, ei0, ei1, zer, out, outd = refs[:6]
            sidx, didx, rows, acc = refs[6:10]
            semg = refs[10:10 + _BUF]
            sems = refs[10 + _BUF]
            degl = refs[11 + _BUF]
        else:
            table, ei0, ei1, zer, out = refs[:5]
            sidx, didx, rows, acc = refs[5:9]
            semg = refs[9:9 + _BUF]
            sems = refs[9 + _BUF]
        c = lax.axis_index("c")
        s = lax.axis_index("s")
        w = s * _NCORES + c

        def deg_update(j):
            ones = jnp.ones((16,), jnp.float32)
            for t in range(_CH // 16):
                dvec = didx[j, pl.ds(t * 16, 16)]
                plsc.addupdate_scatter(degl, [dvec], ones)

        def drain_scatter(r):
            pltpu.make_async_copy(rows.at[r], acc.at[didx.at[0]], sems).wait()

        def fire_half(h):
            return [pltpu.async_copy(
                table.at[sidx.at[h * _BUF + r]], rows.at[r], semg[r])
                for r in range(_BUF)]

        def finish_half(h, handles):
            if with_deg:
                for r in range(_BUF):
                    deg_update(h * _BUF + r)
            for r in range(_BUF):
                handles[r].wait()
                pltpu.async_copy(rows.at[r],
                                 acc.at[didx.at[h * _BUF + r]], sems,
                                 add=True)

        def run_half(h):
            for r in range(_BUF):
                drain_scatter(r)
            finish_half(h, fire_half(h))

        pltpu.sync_copy(ei0.at[pl.ds(w * blkch, blkch)], sidx)
        pltpu.sync_copy(ei1.at[pl.ds(w * blkch, blkch)], didx)
        handles0 = fire_half(0)
        pltpu.sync_copy(zer.at[pl.ds(s * _RPT, _RPT)],
                        acc.at[pl.ds(s * _RPT, _RPT)])
        if with_deg:
            def zbody(i, carry):
                for u in range(8):
                    degl[pl.ds(i * 128 + u * 16, 16)] = jnp.zeros(
                        (16,), jnp.float32)
                return carry
            lax.fori_loop(0, _NPAD // 128, zbody, 0)
        plsc.subcore_barrier()
        finish_half(0, handles0)
        for h in range(1, blkch // _BUF):
            run_half(h)

        def block(k, carry):
            row0 = (w + k * _NW) * blkch
            for r in range(_BUF):
                drain_scatter(r)
            pltpu.sync_copy(ei0.at[pl.ds(row0, blkch)], sidx)
            pltpu.sync_copy(ei1.at[pl.ds(row0, blkch)], didx)
            finish_half(0, fire_half(0))
            for h in range(1, blkch // _BUF):
                run_half(h)
            return carry

        nblk = (nblk_all - w + (_NW - 1)) // _NW
        lax.fori_loop(1, nblk, block, 0)
        for r in range(_BUF):
            drain_scatter(r)

        @pl.when(w == _NW - 1)
        def _():
            row0 = nblk_all * blkch
            pltpu.sync_copy(ei0.at[pl.ds(row0, extra)],
                            sidx.at[pl.ds(0, extra)])
            pltpu.sync_copy(ei1.at[pl.ds(row0, extra)],
                            didx.at[pl.ds(0, extra)])
            for j in range(extra):
                r = j % _BUF
                pltpu.async_copy(table.at[sidx.at[j]], rows.at[r],
                                 semg[r]).wait()
                pltpu.sync_copy(rows.at[r], acc.at[didx.at[j]], add=True)
                if with_deg:
                    deg_update(j)

        if with_deg:
            pltpu.sync_copy(degl, outd.at[w])
        plsc.subcore_barrier()
        pltpu.sync_copy(acc.at[pl.ds(s * _RPT, _RPT)],
                        out.at[c, pl.ds(s * _RPT, _RPT)])

    return seg


_segsum_deg = _make_segsum(True, 24)
_segsum = _make_segsum(False, 32)


def kernel(x, edge_index, id_index, extra, W_self_0, W_id_0, W_nb_0, b_0,
           W_self_1, W_id_1, W_nb_1, b_1, W_mlp1, b_mlp1, W_mlp2, b_mlp2):
    f32 = jnp.float32
    ids_pad = jnp.concatenate(
        [id_index.astype(jnp.int32),
         jnp.full((_NIDP - _NID,), -1, jnp.int32)]).reshape(8, 128)
    wc0 = jnp.concatenate([W_self_0, W_id_0], axis=1)
    wc1 = jnp.concatenate([W_self_1, W_id_1, W_nb_1], axis=1)
    wm2 = jnp.pad(W_mlp2, ((0, 0), (0, 128 - _C)))
    bm2 = jnp.pad(b_mlp2, (0, 128 - _C))[None, :]
    b0 = b_0[None, :]
    b1 = b_1[None, :]
    bm1 = b_mlp1[None, :]
    z_h = jnp.zeros((_NPAD, _H), f32)
    ei0 = edge_index[0].reshape(_NCHUNK, _CH)
    ei1 = edge_index[1].reshape(_NCHUNK, _CH)

    grid = (_N // _BLK,)

    m0 = pl.pallas_call(
        _stage_a1,
        grid=grid,
        in_specs=[
            pl.BlockSpec((_BLK, _D), lambda i: (i, 0)),
            pl.BlockSpec((_D, _H), lambda i: (0, 0)),
        ],
        out_specs=pl.BlockSpec((_BLK, _H), lambda i: (i, 0)),
        out_shape=jax.ShapeDtypeStruct((_N, _H), f32),
    )(x, W_nb_0)

    part0, degp = _segsum_deg(m0, ei0, ei1, z_h)
    degt = jnp.transpose(degp)

    self0, maskf = pl.pallas_call(
        _stage_a2,
        grid=grid,
        in_specs=[
            pl.BlockSpec((_BLK, _D), lambda i: (i, 0)),
            pl.BlockSpec((_D, 2 * _H), lambda i: (0, 0)),
            pl.BlockSpec((8, 128), lambda i: (0, 0)),
        ],
        out_specs=[
            pl.BlockSpec((_BLK, _H), lambda i: (i, 0)),
            pl.BlockSpec((_BLK, 1), lambda i: (i, 0)),
        ],
        out_shape=[
            jax.ShapeDtypeStruct((_N, _H), f32),
            jax.ShapeDtypeStruct((_N, 1), f32),
        ],
    )(x, wc0, ids_pad)

    self1, m1, dinv = pl.pallas_call(
        _stage_b,
        grid=grid,
        in_specs=[
            pl.BlockSpec((_BLK, _H), lambda i: (i, 0)),
            pl.BlockSpec((_NCORES, _BLK, _H), lambda i: (0, i, 0)),
            pl.BlockSpec((_BLK, _NCORES * _NSUB), lambda i: (i, 0)),
            pl.BlockSpec((_BLK, 1), lambda i: (i, 0)),
            pl.BlockSpec((1, _H), lambda i: (0, 0)),
            pl.BlockSpec((_H, 3 * _H), lambda i: (0, 0)),
        ],
        out_specs=[
            pl.BlockSpec((_BLK, _H), lambda i: (i, 0)),
            pl.BlockSpec((_BLK, _H), lambda i: (i, 0)),
            pl.BlockSpec((_BLK, 1), lambda i: (i, 0)),
        ],
        out_shape=[
            jax.ShapeDtypeStruct((_N, _H), f32),
            jax.ShapeDtypeStruct((_N, _H), f32),
            jax.ShapeDtypeStruct((_N, 1), f32),
        ],
    )(self0, part0, degt, maskf, b0, wc1)

    part1 = _segsum(m1, ei0, ei1, z_h)

    outp = pl.pallas_call(
        _stage_d,
        grid=grid,
        in_specs=[
            pl.BlockSpec((_BLK, _H), lambda i: (i, 0)),
            pl.BlockSpec((_NCORES, _BLK, _H), lambda i: (0, i, 0)),
            pl.BlockSpec((_BLK, 1), lambda i: (i, 0)),
            pl.BlockSpec((1, _H), lambda i: (0, 0)),
            pl.BlockSpec((_H, _MLP_H), lambda i: (0, 0)),
            pl.BlockSpec((1, _MLP_H), lambda i: (0, 0)),
            pl.BlockSpec((_MLP_H, 128), lambda i: (0, 0)),
            pl.BlockSpec((1, 128), lambda i: (0, 0)),
        ],
        out_specs=pl.BlockSpec((_BLK, 128), lambda i: (i, 0)),
        out_shape=jax.ShapeDtypeStruct((_N, 128), f32),
    )(self1, part1, dinv, b1, W_mlp1, bm1, wm2, bm2)

    return outp[:, :_C]


# speedup vs baseline: 1.1291x; 1.1291x over previous
import functools

import jax
import jax.numpy as jnp
from jax import lax
from jax.experimental import pallas as pl
from jax.experimental.pallas import tpu as pltpu
from jax.experimental.pallas import tpu_sc as plsc

_N = 10000
_E = 320000
_D = 128
_H = 128
_NID = 1000
_NIDP = 1024
_MLP_H = 256
_C = 6

_BLK = 1000
_CH = 64
_NCORES = 2
_NSUB = 16
_NPAD = 10240
_RPT = _NPAD // _NSUB


def _stage_a(x_ref, w_ref, ids_ref, self0_ref, m_ref, mask_ref):
    i = pl.program_id(0)
    x = x_ref[...]
    y = jnp.dot(x, w_ref[...], preferred_element_type=jnp.float32)
    s_t = y[:, 0:128]
    i_t = y[:, 128:256]
    rid = lax.broadcasted_iota(jnp.int32, (_BLK, 8, 128), 0) + i * _BLK
    hit = (rid == ids_ref[...][None, :, :]).astype(jnp.float32)
    maskf = jnp.minimum(jnp.sum(hit, axis=(1, 2)), 1.0)[:, None]
    self0_ref[...] = s_t + maskf * (i_t - s_t)
    mask_ref[...] = maskf
    m_ref[...] = y[:, 256:384]


def _stage_b(self0_ref, p_ref, degt_ref, mask_ref, b0_ref, w_ref,
             self1_ref, m1_ref, dinv_ref):
    p = p_ref[...]
    agg = p[0] + p[1]
    deg = jnp.sum(degt_ref[...], axis=1, keepdims=True)
    dinv = 1.0 / jnp.maximum(deg, 1.0)
    h1 = jnp.maximum(self0_ref[...] + agg * dinv + b0_ref[...], 0.0)
    y = jnp.dot(h1, w_ref[...], preferred_element_type=jnp.float32)
    s_t = y[:, 0:128]
    i_t = y[:, 128:256]
    maskf = mask_ref[...]
    self1_ref[...] = s_t + maskf * (i_t - s_t)
    m1_ref[...] = y[:, 256:384]
    dinv_ref[...] = dinv


def _stage_d(self1_ref, p_ref, dinv_ref, b1_ref, wm1_ref, bm1_ref, wm2_ref, bm2_ref, out_ref):
    p = p_ref[...]
    agg = (p[0] + p[1]) * dinv_ref[...]
    h2 = jnp.maximum(self1_ref[...] + agg + b1_ref[...], 0.0)
    t = jnp.dot(h2, wm1_ref[...], preferred_element_type=jnp.float32)
    t = jnp.maximum(t + bm1_ref[...], 0.0)
    out_ref[...] = jnp.dot(t, wm2_ref[...], preferred_element_type=jnp.float32) + bm2_ref[...]


_NCHUNK = _E // _CH
_NW = _NCORES * _NSUB
_BUF = 4
def _make_segsum(with_deg, blkch):
    mesh = plsc.VectorSubcoreMesh(core_axis_name="c", subcore_axis_name="s")
    nblk_all = _NCHUNK // blkch
    extra = _NCHUNK - nblk_all * blkch

    part_t = jax.ShapeDtypeStruct((_NCORES, _NPAD, _H), jnp.float32)
    out_type = [part_t, jax.ShapeDtypeStruct((_NW, _NPAD), jnp.float32)] if with_deg else part_t
    scratch = [
        pltpu.VMEM((blkch, _CH), jnp.int32),
        pltpu.VMEM((blkch, _CH), jnp.int32),
        pltpu.VMEM((_BUF, _CH, _H), jnp.float32),
        pltpu.VMEM_SHARED((_NPAD, _H), jnp.float32),
    ] + [pltpu.SemaphoreType.DMA] * (_BUF + 1)
    if with_deg:
        scratch.append(pltpu.VMEM((_NPAD,), jnp.float32))

    @functools.partial(
        pl.kernel, out_type=out_type, mesh=mesh, scratch_types=scratch,
        compiler_params=pltpu.CompilerParams(needs_layout_passes=False))
    def seg(*refs):
        if with_deg:
            table, ei0, ei1, zer, out, outd = refs[:6]
            sidx, didx, rows, acc = refs[6:10]
            semg = refs[10:10 + _BUF]
            sems = refs[10 + _BUF]
            degl = refs[11 + _BUF]
        else:
            table, ei0, ei1, zer, out = refs[:5]
            sidx, didx, rows, acc = refs[5:9]
            semg = refs[9:9 + _BUF]
            sems = refs[9 + _BUF]
        c = lax.axis_index("c")
        s = lax.axis_index("s")
        w = s * _NCORES + c
        pltpu.sync_copy(zer.at[pl.ds(s * _RPT, _RPT)],
                        acc.at[pl.ds(s * _RPT, _RPT)])
        if with_deg:
            def zbody(i, carry):
                for u in range(8):
                    degl[pl.ds(i * 128 + u * 16, 16)] = jnp.zeros(
                        (16,), jnp.float32)
                return carry
            lax.fori_loop(0, _NPAD // 128, zbody, 0)
        plsc.subcore_barrier()

        def deg_update(j):
            ones = jnp.ones((16,), jnp.float32)
            for t in range(_CH // 16):
                dvec = didx[j, pl.ds(t * 16, 16)]
                plsc.addupdate_scatter(degl, [dvec], ones)

        def drain_scatter(r):
            pltpu.make_async_copy(rows.at[r], acc.at[didx.at[0]], sems).wait()

        def block(k, carry):
            b = w + k * _NW
            row0 = b * blkch
            @pl.when(k > 0)
            def _():
                for r in range(_BUF):
                    drain_scatter(r)
            pltpu.sync_copy(ei0.at[pl.ds(row0, blkch)], sidx)
            pltpu.sync_copy(ei1.at[pl.ds(row0, blkch)], didx)
            for h in range(blkch // _BUF):
                handles = []
                for r in range(_BUF):
                    j = h * _BUF + r
                    if h > 0:
                        drain_scatter(r)
                    handles.append(pltpu.async_copy(
                        table.at[sidx.at[j]], rows.at[r], semg[r]))
                if with_deg:
                    for r in range(_BUF):
                        deg_update(h * _BUF + r)
                for r in range(_BUF):
                    handles[r].wait()
                    pltpu.async_copy(rows.at[r],
                                     acc.at[didx.at[h * _BUF + r]], sems,
                                     add=True)
            return carry

        nblk = (nblk_all - w + (_NW - 1)) // _NW
        lax.fori_loop(0, nblk, block, 0)
        @pl.when(nblk > 0)
        def _():
            for r in range(_BUF):
                drain_scatter(r)

        @pl.when(w == _NW - 1)
        def _():
            row0 = nblk_all * blkch
            pltpu.sync_copy(ei0.at[pl.ds(row0, extra)],
                            sidx.at[pl.ds(0, extra)])
            pltpu.sync_copy(ei1.at[pl.ds(row0, extra)],
                            didx.at[pl.ds(0, extra)])
            for j in range(extra):
                r = j % _BUF
                pltpu.async_copy(table.at[sidx.at[j]], rows.at[r],
                                 semg[r]).wait()
                pltpu.sync_copy(rows.at[r], acc.at[didx.at[j]], add=True)
                if with_deg:
                    deg_update(j)

        if with_deg:
            pltpu.sync_copy(degl, outd.at[w])
        plsc.subcore_barrier()
        pltpu.sync_copy(acc.at[pl.ds(s * _RPT, _RPT)],
                        out.at[c, pl.ds(s * _RPT, _RPT)])

    return seg


_segsum_deg = _make_segsum(True, 24)
_segsum = _make_segsum(False, 32)


def kernel(x, edge_index, id_index, extra, W_self_0, W_id_0, W_nb_0, b_0,
           W_self_1, W_id_1, W_nb_1, b_1, W_mlp1, b_mlp1, W_mlp2, b_mlp2):
    f32 = jnp.float32
    ids_pad = jnp.concatenate(
        [id_index.astype(jnp.int32),
         jnp.full((_NIDP - _NID,), -1, jnp.int32)]).reshape(8, 128)
    wc0 = jnp.concatenate([W_self_0, W_id_0, W_nb_0], axis=1)
    wc1 = jnp.concatenate([W_self_1, W_id_1, W_nb_1], axis=1)
    wm2 = jnp.pad(W_mlp2, ((0, 0), (0, 128 - _C)))
    bm2 = jnp.pad(b_mlp2, (0, 128 - _C))[None, :]
    b0 = b_0[None, :]
    b1 = b_1[None, :]
    bm1 = b_mlp1[None, :]
    z_h = jnp.zeros((_NPAD, _H), f32)
    ei0 = edge_index[0].reshape(_NCHUNK, _CH)
    ei1 = edge_index[1].reshape(_NCHUNK, _CH)

    grid = (_N // _BLK,)

    self0, m0, maskf = pl.pallas_call(
        _stage_a,
        grid=grid,
        in_specs=[
            pl.BlockSpec((_BLK, _D), lambda i: (i, 0)),
            pl.BlockSpec((_D, 3 * _H), lambda i: (0, 0)),
            pl.BlockSpec((8, 128), lambda i: (0, 0)),
        ],
        out_specs=[
            pl.BlockSpec((_BLK, _H), lambda i: (i, 0)),
            pl.BlockSpec((_BLK, _H), lambda i: (i, 0)),
            pl.BlockSpec((_BLK, 1), lambda i: (i, 0)),
        ],
        out_shape=[
            jax.ShapeDtypeStruct((_N, _H), f32),
            jax.ShapeDtypeStruct((_N, _H), f32),
            jax.ShapeDtypeStruct((_N, 1), f32),
        ],
    )(x, wc0, ids_pad)

    part0, degp = _segsum_deg(m0, ei0, ei1, z_h)
    degt = jnp.transpose(degp)

    self1, m1, dinv = pl.pallas_call(
        _stage_b,
        grid=grid,
        in_specs=[
            pl.BlockSpec((_BLK, _H), lambda i: (i, 0)),
            pl.BlockSpec((_NCORES, _BLK, _H), lambda i: (0, i, 0)),
            pl.BlockSpec((_BLK, _NCORES * _NSUB), lambda i: (i, 0)),
            pl.BlockSpec((_BLK, 1), lambda i: (i, 0)),
            pl.BlockSpec((1, _H), lambda i: (0, 0)),
            pl.BlockSpec((_H, 3 * _H), lambda i: (0, 0)),
        ],
        out_specs=[
            pl.BlockSpec((_BLK, _H), lambda i: (i, 0)),
            pl.BlockSpec((_BLK, _H), lambda i: (i, 0)),
            pl.BlockSpec((_BLK, 1), lambda i: (i, 0)),
        ],
        out_shape=[
            jax.ShapeDtypeStruct((_N, _H), f32),
            jax.ShapeDtypeStruct((_N, _H), f32),
            jax.ShapeDtypeStruct((_N, 1), f32),
        ],
    )(self0, part0, degt, maskf, b0, wc1)

    part1 = _segsum(m1, ei0, ei1, z_h)

    outp = pl.pallas_call(
        _stage_d,
        grid=grid,
        in_specs=[
            pl.BlockSpec((_BLK, _H), lambda i: (i, 0)),
            pl.BlockSpec((_NCORES, _BLK, _H), lambda i: (0, i, 0)),
            pl.BlockSpec((_BLK, 1), lambda i: (i, 0)),
            pl.BlockSpec((1, _H), lambda i: (0, 0)),
            pl.BlockSpec((_H, _MLP_H), lambda i: (0, 0)),
            pl.BlockSpec((1, _MLP_H), lambda i: (0, 0)),
            pl.BlockSpec((_MLP_H, 128), lambda i: (0, 0)),
            pl.BlockSpec((1, 128), lambda i: (0, 0)),
        ],
        out_specs=pl.BlockSpec((_BLK, 128), lambda i: (i, 0)),
        out_shape=jax.ShapeDtypeStruct((_N, 128), f32),
    )(self1, part1, dinv, b1, W_mlp1, bm1, wm2, bm2)

    return outp[:, :_C]


# speedup vs baseline: 1.1647x; 1.0315x over previous
import functools

import jax
import jax.numpy as jnp
from jax import lax
from jax.experimental import pallas as pl
from jax.experimental.pallas import tpu as pltpu
from jax.experimental.pallas import tpu_sc as plsc

_N = 10000
_E = 320000
_D = 128
_H = 128
_NID = 1000
_NIDP = 1024
_MLP_H = 256
_C = 6

_BLK = 1000
_CH = 128
_NCORES = 2
_NSUB = 16
_NPAD = 10240
_RPT = _NPAD // _NSUB


def _stage_a(x_ref, w_ref, ids_ref, self0_ref, m_ref, mask_ref):
    i = pl.program_id(0)
    x = x_ref[...]
    y = jnp.dot(x, w_ref[...], preferred_element_type=jnp.float32)
    s_t = y[:, 0:128]
    i_t = y[:, 128:256]
    rid = lax.broadcasted_iota(jnp.int32, (_BLK, 8, 128), 0) + i * _BLK
    hit = (rid == ids_ref[...][None, :, :]).astype(jnp.float32)
    maskf = jnp.minimum(jnp.sum(hit, axis=(1, 2)), 1.0)[:, None]
    self0_ref[...] = s_t + maskf * (i_t - s_t)
    mask_ref[...] = maskf
    m_ref[...] = y[:, 256:384]


def _stage_b(self0_ref, p_ref, degt_ref, mask_ref, b0_ref, w_ref,
             self1_ref, m1_ref, dinv_ref):
    p = p_ref[...]
    agg = p[0] + p[1]
    deg = jnp.sum(degt_ref[...], axis=1, keepdims=True)
    dinv = 1.0 / jnp.maximum(deg, 1.0)
    h1 = jnp.maximum(self0_ref[...] + agg * dinv + b0_ref[...], 0.0)
    y = jnp.dot(h1, w_ref[...], preferred_element_type=jnp.float32)
    s_t = y[:, 0:128]
    i_t = y[:, 128:256]
    maskf = mask_ref[...]
    self1_ref[...] = s_t + maskf * (i_t - s_t)
    m1_ref[...] = y[:, 256:384]
    dinv_ref[...] = dinv


def _stage_d(self1_ref, p_ref, dinv_ref, b1_ref, wm1_ref, bm1_ref, wm2_ref, bm2_ref, out_ref):
    p = p_ref[...]
    agg = (p[0] + p[1]) * dinv_ref[...]
    h2 = jnp.maximum(self1_ref[...] + agg + b1_ref[...], 0.0)
    t = jnp.dot(h2, wm1_ref[...], preferred_element_type=jnp.float32)
    t = jnp.maximum(t + bm1_ref[...], 0.0)
    out_ref[...] = jnp.dot(t, wm2_ref[...], preferred_element_type=jnp.float32) + bm2_ref[...]


_NCHUNK = _E // _CH
_NW = _NCORES * _NSUB
_BUF = 2
def _make_segsum(with_deg, blkch):
    mesh = plsc.VectorSubcoreMesh(core_axis_name="c", subcore_axis_name="s")
    nblk_all = _NCHUNK // blkch
    extra = _NCHUNK - nblk_all * blkch

    part_t = jax.ShapeDtypeStruct((_NCORES, _NPAD, _H), jnp.float32)
    out_type = [part_t, jax.ShapeDtypeStruct((_NW, _NPAD), jnp.float32)] if with_deg else part_t
    scratch = [
        pltpu.VMEM((blkch, _CH), jnp.int32),
        pltpu.VMEM((blkch, _CH), jnp.int32),
        pltpu.VMEM((_BUF, _CH, _H), jnp.float32),
        pltpu.VMEM_SHARED((_NPAD, _H), jnp.float32),
    ] + [pltpu.SemaphoreType.DMA] * (_BUF + 1)
    if with_deg:
        scratch.append(pltpu.VMEM((_NPAD,), jnp.float32))

    @functools.partial(
        pl.kernel, out_type=out_type, mesh=mesh, scratch_types=scratch,
        compiler_params=pltpu.CompilerParams(needs_layout_passes=False))
    def seg(*refs):
        if with_deg:
            table, ei0, ei1, zer, out, outd = refs[:6]
            sidx, didx, rows, acc = refs[6:10]
            semg = refs[10:10 + _BUF]
            sems = refs[10 + _BUF]
            degl = refs[11 + _BUF]
        else:
            table, ei0, ei1, zer, out = refs[:5]
            sidx, didx, rows, acc = refs[5:9]
            semg = refs[9:9 + _BUF]
            sems = refs[9 + _BUF]
        c = lax.axis_index("c")
        s = lax.axis_index("s")
        w = s * _NCORES + c
        pltpu.sync_copy(zer.at[pl.ds(s * _RPT, _RPT)],
                        acc.at[pl.ds(s * _RPT, _RPT)])
        if with_deg:
            def zbody(i, carry):
                for u in range(8):
                    degl[pl.ds(i * 128 + u * 16, 16)] = jnp.zeros(
                        (16,), jnp.float32)
                return carry
            lax.fori_loop(0, _NPAD // 128, zbody, 0)
        plsc.subcore_barrier()

        def deg_update(j):
            ones = jnp.ones((16,), jnp.float32)
            for t in range(_CH // 16):
                dvec = didx[j, pl.ds(t * 16, 16)]
                plsc.addupdate_scatter(degl, [dvec], ones)

        def drain_scatter(r):
            pltpu.make_async_copy(rows.at[r], acc.at[didx.at[0]], sems).wait()

        def block(k, carry):
            b = w + k * _NW
            row0 = b * blkch
            @pl.when(k > 0)
            def _():
                for r in range(_BUF):
                    drain_scatter(r)
            pltpu.sync_copy(ei0.at[pl.ds(row0, blkch)], sidx)
            pltpu.sync_copy(ei1.at[pl.ds(row0, blkch)], didx)
            for h in range(blkch // _BUF):
                handles = []
                for r in range(_BUF):
                    j = h * _BUF + r
                    if h > 0:
                        drain_scatter(r)
                    handles.append(pltpu.async_copy(
                        table.at[sidx.at[j]], rows.at[r], semg[r]))
                if with_deg:
                    for r in range(_BUF):
                        deg_update(h * _BUF + r)
                for r in range(_BUF):
                    handles[r].wait()
                    pltpu.async_copy(rows.at[r],
                                     acc.at[didx.at[h * _BUF + r]], sems,
                                     add=True)
            return carry

        nblk = (nblk_all - w + (_NW - 1)) // _NW
        lax.fori_loop(0, nblk, block, 0)
        @pl.when(nblk > 0)
        def _():
            for r in range(_BUF):
                drain_scatter(r)

        @pl.when(w == _NW - 1)
        def _():
            row0 = nblk_all * blkch
            pltpu.sync_copy(ei0.at[pl.ds(row0, extra)],
                            sidx.at[pl.ds(0, extra)])
            pltpu.sync_copy(ei1.at[pl.ds(row0, extra)],
                            didx.at[pl.ds(0, extra)])
            for j in range(extra):
                r = j % _BUF
                pltpu.async_copy(table.at[sidx.at[j]], rows.at[r],
                                 semg[r]).wait()
                pltpu.sync_copy(rows.at[r], acc.at[didx.at[j]], add=True)
                if with_deg:
                    deg_update(j)

        if with_deg:
            pltpu.sync_copy(degl, outd.at[w])
        plsc.subcore_barrier()
        pltpu.sync_copy(acc.at[pl.ds(s * _RPT, _RPT)],
                        out.at[c, pl.ds(s * _RPT, _RPT)])

    return seg


_segsum_deg = _make_segsum(True, 16)
_segsum = _make_segsum(False, 16)


def kernel(x, edge_index, id_index, extra, W_self_0, W_id_0, W_nb_0, b_0,
           W_self_1, W_id_1, W_nb_1, b_1, W_mlp1, b_mlp1, W_mlp2, b_mlp2):
    f32 = jnp.float32
    ids_pad = jnp.concatenate(
        [id_index.astype(jnp.int32),
         jnp.full((_NIDP - _NID,), -1, jnp.int32)]).reshape(8, 128)
    wc0 = jnp.concatenate([W_self_0, W_id_0, W_nb_0], axis=1)
    wc1 = jnp.concatenate([W_self_1, W_id_1, W_nb_1], axis=1)
    wm2 = jnp.pad(W_mlp2, ((0, 0), (0, 128 - _C)))
    bm2 = jnp.pad(b_mlp2, (0, 128 - _C))[None, :]
    b0 = b_0[None, :]
    b1 = b_1[None, :]
    bm1 = b_mlp1[None, :]
    z_h = jnp.zeros((_NPAD, _H), f32)
    ei0 = edge_index[0].reshape(_NCHUNK, _CH)
    ei1 = edge_index[1].reshape(_NCHUNK, _CH)

    grid = (_N // _BLK,)

    self0, m0, maskf = pl.pallas_call(
        _stage_a,
        grid=grid,
        in_specs=[
            pl.BlockSpec((_BLK, _D), lambda i: (i, 0)),
            pl.BlockSpec((_D, 3 * _H), lambda i: (0, 0)),
            pl.BlockSpec((8, 128), lambda i: (0, 0)),
        ],
        out_specs=[
            pl.BlockSpec((_BLK, _H), lambda i: (i, 0)),
            pl.BlockSpec((_BLK, _H), lambda i: (i, 0)),
            pl.BlockSpec((_BLK, 1), lambda i: (i, 0)),
        ],
        out_shape=[
            jax.ShapeDtypeStruct((_N, _H), f32),
            jax.ShapeDtypeStruct((_N, _H), f32),
            jax.ShapeDtypeStruct((_N, 1), f32),
        ],
    )(x, wc0, ids_pad)

    part0, degp = _segsum_deg(m0, ei0, ei1, z_h)
    degt = jnp.transpose(degp)

    self1, m1, dinv = pl.pallas_call(
        _stage_b,
        grid=grid,
        in_specs=[
            pl.BlockSpec((_BLK, _H), lambda i: (i, 0)),
            pl.BlockSpec((_NCORES, _BLK, _H), lambda i: (0, i, 0)),
            pl.BlockSpec((_BLK, _NCORES * _NSUB), lambda i: (i, 0)),
            pl.BlockSpec((_BLK, 1), lambda i: (i, 0)),
            pl.BlockSpec((1, _H), lambda i: (0, 0)),
            pl.BlockSpec((_H, 3 * _H), lambda i: (0, 0)),
        ],
        out_specs=[
            pl.BlockSpec((_BLK, _H), lambda i: (i, 0)),
            pl.BlockSpec((_BLK, _H), lambda i: (i, 0)),
            pl.BlockSpec((_BLK, 1), lambda i: (i, 0)),
        ],
        out_shape=[
            jax.ShapeDtypeStruct((_N, _H), f32),
            jax.ShapeDtypeStruct((_N, _H), f32),
            jax.ShapeDtypeStruct((_N, 1), f32),
        ],
    )(self0, part0, degt, maskf, b0, wc1)

    part1 = _segsum(m1, ei0, ei1, z_h)

    outp = pl.pallas_call(
        _stage_d,
        grid=grid,
        in_specs=[
            pl.BlockSpec((_BLK, _H), lambda i: (i, 0)),
            pl.BlockSpec((_NCORES, _BLK, _H), lambda i: (0, i, 0)),
            pl.BlockSpec((_BLK, 1), lambda i: (i, 0)),
            pl.BlockSpec((1, _H), lambda i: (0, 0)),
            pl.BlockSpec((_H, _MLP_H), lambda i: (0, 0)),
            pl.BlockSpec((1, _MLP_H), lambda i: (0, 0)),
            pl.BlockSpec((_MLP_H, 128), lambda i: (0, 0)),
            pl.BlockSpec((1, 128), lambda i: (0, 0)),
        ],
        out_specs=pl.BlockSpec((_BLK, 128), lambda i: (i, 0)),
        out_shape=jax.ShapeDtypeStruct((_N, 128), f32),
    )(self1, part1, dinv, b1, W_mlp1, bm1, wm2, bm2)

    return outp[:, :_C]
